# baseline (speedup 1.0000x reference)
import functools

import numpy as np
import jax
import jax.numpy as jnp
from jax import lax
from jax.experimental import pallas as pl
from jax.experimental.pallas import tpu as pltpu
from jax.experimental.pallas import tpu_sc as plsc

N = 512
NK = 2 * (N - 1)
NKP = 1024
NSEG = 426
NSEGP = 512
NF = 16
EPS = 1e-8
DUMP = N * N
QROWS = 128
QWORDS = QROWS * N

_f = np.fft.fftfreq(N).astype(np.float64)
_c = np.fft.fft(_f * _f).real
_i = np.arange(N)
_M_NP = _c[(_i[:, None] - _i[None, :]) % N].astype(np.float32)


def _coords_body(p1x_r, p1y_r, p2x_r, p2y_r, segf_r, out_ref):
    p1x, p1y, p2x, p2y, segf = (
        p1x_r[0], p1y_r[0], p2x_r[0], p2y_r[0], segf_r[0],
    )
    x1 = jnp.floor(p1x * (N - 1)).astype(jnp.int32)
    y1 = jnp.floor(p1y * (N - 1)).astype(jnp.int32)
    x2 = jnp.floor(p2x * (N - 1)).astype(jnp.int32)
    y2 = jnp.floor(p2y * (N - 1)).astype(jnp.int32)
    in01 = (
        (p1x >= 0) & (p1x <= 1) & (p1y >= 0) & (p1y <= 1)
        & (p2x >= 0) & (p2x <= 1) & (p2y >= 0) & (p2y <= 1)
    )
    ok = in01 & (segf > 0)
    d = jnp.maximum(2 * jnp.maximum(jnp.abs(x2 - x1), jnp.abs(y2 - y1)), 2) - 1
    d2 = 2 * d
    rec = 1.0 / d2.astype(jnp.float32)
    k = lax.broadcasted_iota(jnp.int32, (NKP, NSEGP), 0)
    kk = jnp.minimum(k, d)

    def interp(a1, a2):
        num = 2 * (a1 * (d - kk) + a2 * kk) + d
        q0 = jnp.floor(num.astype(jnp.float32) * rec).astype(jnp.int32)
        r = num - q0 * d2
        return q0 + (r >= d2).astype(jnp.int32) - (r < 0).astype(jnp.int32)

    lx = interp(x1, x2)
    ly = interp(y1, y2)
    lin = ly * N + lx
    out_ref[0] = jnp.where(ok, lin, DUMP)


def _coords(p1x, p1y, p2x, p2y, segf):
    spec = pl.BlockSpec((1, 1, NSEGP), lambda i: (i, 0, 0))
    return pl.pallas_call(
        _coords_body,
        grid=(NF,),
        in_specs=[spec] * 5,
        out_specs=pl.BlockSpec((1, NKP, NSEGP), lambda i: (i, 0, 0)),
        out_shape=jax.ShapeDtypeStruct((NF, NKP, NSEGP), jnp.int32),
    )(p1x, p1y, p2x, p2y, segf)


KROWS = 32
NCH = NKP // KROWS
NSEG_SCAN = 432
JV = NSEG_SCAN // 16


def _scatter_fields(idx, nv):
    mesh = plsc.VectorSubcoreMesh(core_axis_name="c", subcore_axis_name="s")

    @functools.partial(
        pl.kernel,
        mesh=mesh,
        out_type=jax.ShapeDtypeStruct((NF, 4, QWORDS), jnp.float32),
        scratch_types=[
            pltpu.VMEM((KROWS, NSEGP), jnp.int32),
            pltpu.VMEM((KROWS, NSEGP), jnp.int32),
            pltpu.VMEM((QWORDS,), jnp.float32),
            pltpu.VMEM((NCH,), jnp.int32),
            pltpu.SemaphoreType.DMA,
            pltpu.SemaphoreType.DMA,
        ],
        compiler_params=pltpu.CompilerParams(needs_layout_passes=False),
    )
    def scat(idx_hbm, nv_hbm, out_hbm, buf0, buf1, field, nvbuf, sem0, sem1):
        wid = lax.axis_index("s") * 2 + lax.axis_index("c")
        ones = jnp.full((16,), 1.0, jnp.float32)
        zeros16 = jnp.zeros((16,), jnp.float32)
        qwords_u = jnp.uint32(QWORDS)
        lanes = jnp.arange(16, dtype=jnp.int32)
        for p in range(2):
            slot = wid + 32 * p
            fld = slot // 4
            base = (slot % 4) * QWORDS

            pltpu.sync_copy(nv_hbm.at[fld], nvbuf)
            nva = nvbuf[pl.ds(0, 16)]
            nvb = nvbuf[pl.ds(16, 16)]

            @plsc.parallel_loop(0, QWORDS // 16, unroll=8)
            def _(i):
                field[pl.ds(i * 16, 16)] = zeros16

            def src(c):
                return idx_hbm.at[fld, pl.ds(c * KROWS, KROWS)]

            def nv_at(c):
                cfull = jnp.full((16,), c, jnp.int32)
                nvvec = jnp.where(cfull >= 16, nvb, nva)
                sel = jnp.where(lanes == (cfull & 15), nvvec, 0)
                return lax.reduce_max(sel, (0,))

            pltpu.async_copy(src(0), buf0, sem0)
            pltpu.async_copy(src(1), buf1, sem1)

            def do_chunk(c, buf, sem):
                pltpu.make_async_copy(src(c), buf, sem).wait()
                njv = nv_at(c)
                for r in range(KROWS):
                    @plsc.parallel_loop(0, njv, unroll=3)
                    def _(j):
                        idxv = buf[r, pl.ds(j * 16, 16)]
                        loc = idxv - base
                        msk = plsc.bitcast(loc, jnp.uint32) < qwords_u
                        plsc.store_scatter(field, [loc], ones, mask=msk)

                @pl.when(c + 2 < NCH)
                def _():
                    pltpu.async_copy(src(c + 2), buf, sem)

            def pair_body(t, _):
                do_chunk(2 * t, buf0, sem0)
                do_chunk(2 * t + 1, buf1, sem1)
                return 0

            lax.fori_loop(0, NCH // 2, pair_body, 0)
            pltpu.sync_copy(field, out_hbm.at[fld, slot % 4])

    return scat(idx, nv)


def _dilate3(x):
    z_row = jnp.zeros((1, N), jnp.float32)
    up = jnp.concatenate([x[1:], z_row], axis=0)
    dn = jnp.concatenate([z_row, x[:-1]], axis=0)
    v = jnp.maximum(x, jnp.maximum(up, dn))
    z_col = jnp.zeros((N, 1), jnp.float32)
    lf = jnp.concatenate([v[:, 1:], z_col], axis=1)
    rt = jnp.concatenate([z_col, v[:, :-1]], axis=1)
    return jnp.maximum(v, jnp.maximum(lf, rt))


def _loss_body(gt_ref, pr_ref, m_ref, out_ref):
    b = pl.program_id(0)
    g = _dilate3(gt_ref[0])
    p = _dilate3(pr_ref[0])
    dd = g - p
    m = m_ref[...]
    q = jnp.dot(m, dd, preferred_element_type=jnp.float32) + jnp.dot(
        dd, m, preferred_element_type=jnp.float32
    )
    part = jnp.float32(N) * jnp.sum(q * dd) + jnp.float32(EPS * N * N) * jnp.sum(dd * dd)

    @pl.when(b == 0)
    def _():
        out_ref[0, 0] = 0.0

    out_ref[0, 0] += part

    @pl.when(b == 8 - 1)
    def _():
        out_ref[0, 0] = out_ref[0, 0] * jnp.float32(1.0 / (float(N) ** 4) / 8.0)


def _spectral_loss(fields, m):
    return pl.pallas_call(
        _loss_body,
        grid=(8,),
        in_specs=[
            pl.BlockSpec((1, N, N), lambda b: (b, 0, 0)),
            pl.BlockSpec((1, N, N), lambda b: (b + 8, 0, 0)),
            pl.BlockSpec((N, N), lambda b: (0, 0)),
        ],
        out_specs=pl.BlockSpec(memory_space=pltpu.SMEM),
        out_shape=jax.ShapeDtypeStruct((1, 1), jnp.float32),
    )(fields, fields, m)


def kernel(pred_keypoints, gt_keypoints, valid_mask):
    kp = jnp.concatenate([gt_keypoints, pred_keypoints], axis=0)
    vm = jnp.concatenate([valid_mask, valid_mask], axis=0)

    p1 = kp[:, :, :-1, :].reshape(NF, NSEG, 2)
    p2 = kp[:, :, 1:, :].reshape(NF, NSEG, 2)
    segv = vm[:, :, :-1] & vm[:, :, 1:] & jnp.any(vm, axis=2)[:, :, None]
    segv = segv.reshape(NF, NSEG)
    segf = segv.astype(jnp.float32)

    xi = jnp.floor(kp[..., 0] * (N - 1)).astype(jnp.int32)
    yi = jnp.floor(kp[..., 1] * (N - 1)).astype(jnp.int32)
    dx = jnp.abs(xi[:, :, 1:] - xi[:, :, :-1]).reshape(NF, NSEG)
    dy = jnp.abs(yi[:, :, 1:] - yi[:, :, :-1]).reshape(NF, NSEG)
    dseg = jnp.maximum(2 * jnp.maximum(dx, dy), 2) - 1
    dkey = jnp.where(segv, dseg, -1)
    order = jnp.argsort(-dkey, axis=1)

    def perm(a):
        return jnp.take_along_axis(a, order, axis=1)

    pad = NSEGP - NSEG

    def padded(a):
        return jnp.pad(a, ((0, 0), (0, pad)))[:, None, :]

    p1x = padded(perm(p1[:, :, 0]))
    p1y = padded(perm(p1[:, :, 1]))
    p2x = padded(perm(p2[:, :, 0]))
    p2y = padded(perm(p2[:, :, 1]))
    segf = padded(perm(segf))

    kmax = jnp.take_along_axis(dkey, order, axis=1)[:, ::16]
    nv = jnp.sum(
        (kmax[:, None, :] >= (jnp.arange(NCH, dtype=jnp.int32) * KROWS)[None, :, None]),
        axis=2,
        dtype=jnp.int32,
    )

    idx = _coords(p1x, p1y, p2x, p2y, segf)
    quarters = _scatter_fields(idx, nv)
    fields = quarters.reshape(NF, N, N)
    m = jnp.asarray(_M_NP)
    out = _spectral_loss(fields, m)
    return out[0, 0]


# speedup vs baseline: 1.3620x; 1.3620x over previous
import functools

import numpy as np
import jax
import jax.numpy as jnp
from jax import lax
from jax.experimental import pallas as pl
from jax.experimental.pallas import tpu as pltpu
from jax.experimental.pallas import tpu_sc as plsc

N = 512
NK = 2 * (N - 1)
NKP = 1024
NSEG = 426
NSEGP = 512
NF = 16
EPS = 1e-8
DUMP = N * N
QROWS = 128
QWORDS = QROWS * N

_f = np.fft.fftfreq(N).astype(np.float64)
_c = np.fft.fft(_f * _f).real
_i = np.arange(N)
_M_NP = _c[(_i[:, None] - _i[None, :]) % N].astype(np.float32)


def _coords_body(p1x_r, p1y_r, p2x_r, p2y_r, segf_r, out_ref):
    p1x, p1y, p2x, p2y, segf = (
        p1x_r[0], p1y_r[0], p2x_r[0], p2y_r[0], segf_r[0],
    )
    x1 = jnp.floor(p1x * (N - 1)).astype(jnp.int32)
    y1 = jnp.floor(p1y * (N - 1)).astype(jnp.int32)
    x2 = jnp.floor(p2x * (N - 1)).astype(jnp.int32)
    y2 = jnp.floor(p2y * (N - 1)).astype(jnp.int32)
    in01 = (
        (p1x >= 0) & (p1x <= 1) & (p1y >= 0) & (p1y <= 1)
        & (p2x >= 0) & (p2x <= 1) & (p2y >= 0) & (p2y <= 1)
    )
    ok = in01 & (segf > 0)
    d = jnp.maximum(2 * jnp.maximum(jnp.abs(x2 - x1), jnp.abs(y2 - y1)), 2) - 1
    d2 = 2 * d
    rec = 1.0 / d2.astype(jnp.float32)
    k = lax.broadcasted_iota(jnp.int32, (NKP, NSEGP), 0)
    kk = jnp.minimum(k, d)

    def interp(a1, a2):
        num = 2 * (a1 * (d - kk) + a2 * kk) + d
        q0 = jnp.floor(num.astype(jnp.float32) * rec).astype(jnp.int32)
        r = num - q0 * d2
        return q0 + (r >= d2).astype(jnp.int32) - (r < 0).astype(jnp.int32)

    lx = interp(x1, x2)
    ly = interp(y1, y2)
    lin = ly * N + lx
    out_ref[0] = jnp.where(ok, lin, DUMP)


def _coords(p1x, p1y, p2x, p2y, segf):
    spec = pl.BlockSpec((1, 1, NSEGP), lambda i: (i, 0, 0))
    return pl.pallas_call(
        _coords_body,
        grid=(NF,),
        in_specs=[spec] * 5,
        out_specs=pl.BlockSpec((1, NKP, NSEGP), lambda i: (i, 0, 0)),
        out_shape=jax.ShapeDtypeStruct((NF, NKP, NSEGP), jnp.int32),
    )(p1x, p1y, p2x, p2y, segf)


KROWS = 32
NCH = NKP // KROWS
NSEG_SCAN = 432
JV = NSEG_SCAN // 16


def _scatter_fields(idx):
    mesh = plsc.VectorSubcoreMesh(core_axis_name="c", subcore_axis_name="s")

    @functools.partial(
        pl.kernel,
        mesh=mesh,
        out_type=jax.ShapeDtypeStruct((NF, 4, QWORDS), jnp.float32),
        scratch_types=[
            pltpu.VMEM((KROWS, NSEGP), jnp.int32),
            pltpu.VMEM((KROWS, NSEGP), jnp.int32),
            pltpu.VMEM((QWORDS,), jnp.float32),
            pltpu.SemaphoreType.DMA,
            pltpu.SemaphoreType.DMA,
        ],
        compiler_params=pltpu.CompilerParams(
            needs_layout_passes=False, use_tc_tiling_on_sc=False
        ),
    )
    def scat(idx_hbm, out_hbm, buf0, buf1, field, sem0, sem1):
        wid = lax.axis_index("s") * 2 + lax.axis_index("c")
        ones = jnp.full((16,), 1.0, jnp.float32)
        zeros16 = jnp.zeros((16,), jnp.float32)
        qwords_u = jnp.uint32(QWORDS)
        for p in range(2):
            slot = wid + 32 * p
            fld = slot // 4
            base = (slot % 4) * QWORDS

            @plsc.parallel_loop(0, QWORDS // 16, unroll=8)
            def _(i):
                field[pl.ds(i * 16, 16)] = zeros16

            def src(c):
                return idx_hbm.at[fld, pl.ds(c * KROWS, KROWS)]

            pltpu.async_copy(src(0), buf0, sem0)
            pltpu.async_copy(src(1), buf1, sem1)

            def do_chunk(c, buf, sem):
                pltpu.make_async_copy(src(c), buf, sem).wait()
                for r in range(KROWS):
                    @plsc.parallel_loop(0, JV, unroll=3)
                    def _(j):
                        idxv = buf[r, pl.ds(j * 16, 16)]
                        loc = idxv - base
                        msk = plsc.bitcast(loc, jnp.uint32) < qwords_u
                        plsc.store_scatter(field, [loc], ones, mask=msk)

                @pl.when(c + 2 < NCH)
                def _():
                    pltpu.async_copy(src(c + 2), buf, sem)

            def pair_body(t, _):
                do_chunk(2 * t, buf0, sem0)
                do_chunk(2 * t + 1, buf1, sem1)
                return 0

            lax.fori_loop(0, NCH // 2, pair_body, 0)
            pltpu.sync_copy(field, out_hbm.at[fld, slot % 4])

    return scat(idx)


def _dilate3(x):
    z_row = jnp.zeros((1, N), jnp.float32)
    up = jnp.concatenate([x[1:], z_row], axis=0)
    dn = jnp.concatenate([z_row, x[:-1]], axis=0)
    v = jnp.maximum(x, jnp.maximum(up, dn))
    z_col = jnp.zeros((N, 1), jnp.float32)
    lf = jnp.concatenate([v[:, 1:], z_col], axis=1)
    rt = jnp.concatenate([z_col, v[:, :-1]], axis=1)
    return jnp.maximum(v, jnp.maximum(lf, rt))


def _loss_body(gt_ref, pr_ref, m_ref, out_ref):
    b = pl.program_id(0)
    g = _dilate3(gt_ref[0])
    p = _dilate3(pr_ref[0])
    dd = g - p
    m = m_ref[...]
    q = jnp.dot(m, dd, preferred_element_type=jnp.float32) + jnp.dot(
        dd, m, preferred_element_type=jnp.float32
    )
    part = jnp.float32(N) * jnp.sum(q * dd) + jnp.float32(EPS * N * N) * jnp.sum(dd * dd)

    @pl.when(b == 0)
    def _():
        out_ref[0, 0] = 0.0

    out_ref[0, 0] += part

    @pl.when(b == 8 - 1)
    def _():
        out_ref[0, 0] = out_ref[0, 0] * jnp.float32(1.0 / (float(N) ** 4) / 8.0)


def _spectral_loss(fields, m):
    return pl.pallas_call(
        _loss_body,
        grid=(8,),
        in_specs=[
            pl.BlockSpec((1, N, N), lambda b: (b, 0, 0)),
            pl.BlockSpec((1, N, N), lambda b: (b + 8, 0, 0)),
            pl.BlockSpec((N, N), lambda b: (0, 0)),
        ],
        out_specs=pl.BlockSpec(memory_space=pltpu.SMEM),
        out_shape=jax.ShapeDtypeStruct((1, 1), jnp.float32),
    )(fields, fields, m)


def kernel(pred_keypoints, gt_keypoints, valid_mask):
    kp = jnp.concatenate([gt_keypoints, pred_keypoints], axis=0)
    vm = jnp.concatenate([valid_mask, valid_mask], axis=0)

    p1 = kp[:, :, :-1, :].reshape(NF, NSEG, 2)
    p2 = kp[:, :, 1:, :].reshape(NF, NSEG, 2)
    segv = vm[:, :, :-1] & vm[:, :, 1:] & jnp.any(vm, axis=2)[:, :, None]
    segf = segv.reshape(NF, NSEG).astype(jnp.float32)

    pad = NSEGP - NSEG

    def padded(a):
        return jnp.pad(a, ((0, 0), (0, pad)))[:, None, :]

    p1x = padded(p1[:, :, 0])
    p1y = padded(p1[:, :, 1])
    p2x = padded(p2[:, :, 0])
    p2y = padded(p2[:, :, 1])
    segf = padded(segf)

    idx = _coords(p1x, p1y, p2x, p2y, segf)
    quarters = _scatter_fields(idx)
    fields = quarters.reshape(NF, N, N)
    m = jnp.asarray(_M_NP)
    out = _spectral_loss(fields, m)
    return out[0, 0]


# speedup vs baseline: 1.5241x; 1.1190x over previous
import functools

import numpy as np
import jax
import jax.numpy as jnp
from jax import lax
from jax.experimental import pallas as pl
from jax.experimental.pallas import tpu as pltpu
from jax.experimental.pallas import tpu_sc as plsc

N = 512
NK = 2 * (N - 1)
NKP = 1024
NSEG = 426
NSEGP = 512
NF = 16
EPS = 1e-8
DUMP = N * N
QROWS = 128
QWORDS = QROWS * N

_f = np.fft.fftfreq(N).astype(np.float64)
_c = np.fft.fft(_f * _f).real
_i = np.arange(N)
_M_NP = _c[(_i[:, None] - _i[None, :]) % N].astype(np.float32)


def _coords_body(p1x_r, p1y_r, p2x_r, p2y_r, segf_r, out_ref):
    p1x, p1y, p2x, p2y, segf = (
        p1x_r[0], p1y_r[0], p2x_r[0], p2y_r[0], segf_r[0],
    )
    x1 = jnp.floor(p1x * (N - 1)).astype(jnp.int32)
    y1 = jnp.floor(p1y * (N - 1)).astype(jnp.int32)
    x2 = jnp.floor(p2x * (N - 1)).astype(jnp.int32)
    y2 = jnp.floor(p2y * (N - 1)).astype(jnp.int32)
    in01 = (
        (p1x >= 0) & (p1x <= 1) & (p1y >= 0) & (p1y <= 1)
        & (p2x >= 0) & (p2x <= 1) & (p2y >= 0) & (p2y <= 1)
    )
    ok = in01 & (segf > 0)
    d = jnp.maximum(2 * jnp.maximum(jnp.abs(x2 - x1), jnp.abs(y2 - y1)), 2) - 1
    d2 = 2 * d
    rec = 1.0 / d2.astype(jnp.float32)
    k = lax.broadcasted_iota(jnp.int32, (NKP, NSEGP), 0)
    kk = jnp.minimum(k, d)

    def interp(a1, a2):
        num = 2 * (a1 * (d - kk) + a2 * kk) + d
        q0 = jnp.floor(num.astype(jnp.float32) * rec).astype(jnp.int32)
        r = num - q0 * d2
        return q0 + (r >= d2).astype(jnp.int32) - (r < 0).astype(jnp.int32)

    lx = interp(x1, x2)
    ly = interp(y1, y2)
    lin = ly * N + lx
    out_ref[0] = jnp.where(ok, lin, DUMP)


def _coords(p1x, p1y, p2x, p2y, segf):
    spec = pl.BlockSpec((1, 1, NSEGP), lambda i: (i, 0, 0))
    return pl.pallas_call(
        _coords_body,
        grid=(NF,),
        in_specs=[spec] * 5,
        out_specs=pl.BlockSpec((1, NKP, NSEGP), lambda i: (i, 0, 0)),
        out_shape=jax.ShapeDtypeStruct((NF, NKP, NSEGP), jnp.int32),
    )(p1x, p1y, p2x, p2y, segf)


KROWS = 32
NCH = NKP // KROWS
NSEG_SCAN = 432
JV = NSEG_SCAN // 16


def _scatter_fields(idx):
    mesh = plsc.VectorSubcoreMesh(core_axis_name="c", subcore_axis_name="s")

    @functools.partial(
        pl.kernel,
        mesh=mesh,
        out_type=jax.ShapeDtypeStruct((NF, 4, QWORDS), jnp.float32),
        scratch_types=[
            pltpu.VMEM((KROWS, NSEGP), jnp.int32),
            pltpu.VMEM((KROWS, NSEGP), jnp.int32),
            pltpu.VMEM((QWORDS,), jnp.float32),
            pltpu.SemaphoreType.DMA,
            pltpu.SemaphoreType.DMA,
        ],
        compiler_params=pltpu.CompilerParams(needs_layout_passes=False),
    )
    def scat(idx_hbm, out_hbm, buf0, buf1, field, sem0, sem1):
        wid = lax.axis_index("s") * 2 + lax.axis_index("c")
        ones = jnp.full((16,), 1.0, jnp.float32)
        zeros16 = jnp.zeros((16,), jnp.float32)
        qwords_u = jnp.uint32(QWORDS)
        for p in range(2):
            slot = wid + 32 * p
            fld = slot // 4
            base = (slot % 4) * QWORDS

            @plsc.parallel_loop(0, QWORDS // 16, unroll=8)
            def _(i):
                field[pl.ds(i * 16, 16)] = zeros16

            def src(c):
                return idx_hbm.at[fld, pl.ds(c * KROWS, KROWS)]

            pltpu.async_copy(src(0), buf0, sem0)
            pltpu.async_copy(src(1), buf1, sem1)

            def do_chunk(c, buf, sem):
                pltpu.make_async_copy(src(c), buf, sem).wait()
                for r in range(KROWS):
                    @plsc.parallel_loop(0, JV, unroll=3)
                    def _(j):
                        idxv = buf[r, pl.ds(j * 16, 16)]
                        loc = idxv - base
                        msk = plsc.bitcast(loc, jnp.uint32) < qwords_u
                        plsc.store_scatter(field, [loc], ones, mask=msk)

                @pl.when(c + 2 < NCH)
                def _():
                    pltpu.async_copy(src(c + 2), buf, sem)

            def pair_body(t, _):
                do_chunk(2 * t, buf0, sem0)
                do_chunk(2 * t + 1, buf1, sem1)
                return 0

            lax.fori_loop(0, NCH // 2, pair_body, 0)
            pltpu.sync_copy(field, out_hbm.at[fld, slot % 4])

    return scat(idx)


def _dilate3(x):
    z_row = jnp.zeros((1, N), jnp.float32)
    up = jnp.concatenate([x[1:], z_row], axis=0)
    dn = jnp.concatenate([z_row, x[:-1]], axis=0)
    v = jnp.maximum(x, jnp.maximum(up, dn))
    z_col = jnp.zeros((N, 1), jnp.float32)
    lf = jnp.concatenate([v[:, 1:], z_col], axis=1)
    rt = jnp.concatenate([z_col, v[:, :-1]], axis=1)
    return jnp.maximum(v, jnp.maximum(lf, rt))


def _loss_body(gt_ref, pr_ref, m_ref, out_ref):
    b = pl.program_id(0)
    g = _dilate3(gt_ref[0])
    p = _dilate3(pr_ref[0])
    dd = g - p
    m = m_ref[...]
    q = jnp.dot(m, dd, preferred_element_type=jnp.float32) + jnp.dot(
        dd, m, preferred_element_type=jnp.float32
    )
    part = jnp.float32(N) * jnp.sum(q * dd) + jnp.float32(EPS * N * N) * jnp.sum(dd * dd)

    @pl.when(b == 0)
    def _():
        out_ref[0, 0] = 0.0

    out_ref[0, 0] += part

    @pl.when(b == 8 - 1)
    def _():
        out_ref[0, 0] = out_ref[0, 0] * jnp.float32(1.0 / (float(N) ** 4) / 8.0)


def _spectral_loss(fields, m):
    return pl.pallas_call(
        _loss_body,
        grid=(8,),
        in_specs=[
            pl.BlockSpec((1, N, N), lambda b: (b, 0, 0)),
            pl.BlockSpec((1, N, N), lambda b: (b + 8, 0, 0)),
            pl.BlockSpec((N, N), lambda b: (0, 0)),
        ],
        out_specs=pl.BlockSpec(memory_space=pltpu.SMEM),
        out_shape=jax.ShapeDtypeStruct((1, 1), jnp.float32),
    )(fields, fields, m)


def kernel(pred_keypoints, gt_keypoints, valid_mask):
    kp = jnp.concatenate([gt_keypoints, pred_keypoints], axis=0)
    vm = jnp.concatenate([valid_mask, valid_mask], axis=0)

    p1 = kp[:, :, :-1, :].reshape(NF, NSEG, 2)
    p2 = kp[:, :, 1:, :].reshape(NF, NSEG, 2)
    segv = vm[:, :, :-1] & vm[:, :, 1:] & jnp.any(vm, axis=2)[:, :, None]
    segf = segv.reshape(NF, NSEG).astype(jnp.float32)

    pad = NSEGP - NSEG

    def padded(a):
        return jnp.pad(a, ((0, 0), (0, pad)))[:, None, :]

    p1x = padded(p1[:, :, 0])
    p1y = padded(p1[:, :, 1])
    p2x = padded(p2[:, :, 0])
    p2y = padded(p2[:, :, 1])
    segf = padded(segf)

    idx = _coords(p1x, p1y, p2x, p2y, segf)
    quarters = _scatter_fields(idx)
    fields = quarters.reshape(NF, N, N)
    m = jnp.asarray(_M_NP)
    out = _spectral_loss(fields, m)
    return out[0, 0]
